# Initial kernel scaffold; baseline (speedup 1.0000x reference)
import jax
import jax.numpy as jnp
from jax.experimental import pallas as pl


def kernel(x, edge_index, W1_rel, b1, W1_root, W2_rel, b2, W2_root, W_lin, b_lin):
    raise NotImplementedError("write your pallas kernel here")


# speedup vs baseline: 5.0614x; 5.0614x over previous
import functools

import jax
import jax.numpy as jnp
from jax import lax
from jax.experimental import pallas as pl
from jax.experimental.pallas import tpu as pltpu
from jax.experimental.pallas import tpu_sc as plsc

N = 10000
E = 320000
D = 128

NC = 2
NS = 16
NW = NC * NS
EPW = E // NW
CHUNK = 80
NCHUNK = EPW // CHUNK
N_PAD = 10240
RPT = N_PAD // NS
ZROWS = 128


def _segsum_body(x_hbm, src_hbm, dst_hbm, out_hbm, src_v, dst_v, rows_v, zbuf_v, acc_sh):
    core = lax.axis_index("c")
    sid = lax.axis_index("s")

    def zrow(i, _):
        def zcol(j, _):
            zbuf_v[i, pl.ds(j * 16, 16)] = jnp.zeros((16,), jnp.float32)
            return 0
        return lax.fori_loop(0, D // 16, zcol, 0)
    lax.fori_loop(0, ZROWS, zrow, 0)

    if True:
        for k in range(RPT // ZROWS):
            pltpu.sync_copy(zbuf_v, acc_sh.at[pl.ds(sid * RPT + k * ZROWS, ZROWS)])
        plsc.subcore_barrier()

        base = (core * NS + sid) * EPW

        def body(i, _):
            off = base + i * CHUNK
            pltpu.sync_copy(src_hbm.at[pl.ds(off, CHUNK)], src_v)
            pltpu.sync_copy(dst_hbm.at[pl.ds(off, CHUNK)], dst_v)
            pltpu.sync_copy(x_hbm.at[src_v], rows_v)
            pltpu.sync_copy(rows_v, acc_sh.at[dst_v], add=True)
            return 0
        lax.fori_loop(0, NCHUNK, body, 0)

        plsc.subcore_barrier()
        for k in range(RPT // ZROWS):
            r = sid * RPT + k * ZROWS
            pltpu.sync_copy(acc_sh.at[pl.ds(r, ZROWS)], out_hbm.at[core, pl.ds(r, ZROWS)])


_segsum = pl.kernel(
    _segsum_body,
    out_type=jax.ShapeDtypeStruct((NC, N_PAD, D), jnp.float32),
    mesh=plsc.VectorSubcoreMesh(
        core_axis_name="c", subcore_axis_name="s", num_cores=NC, num_subcores=NS
    ),
    scratch_types=[
        pltpu.VMEM((CHUNK,), jnp.int32),
        pltpu.VMEM((CHUNK,), jnp.int32),
        pltpu.VMEM((CHUNK, D), jnp.float32),
        pltpu.VMEM((ZROWS, D), jnp.float32),
        pltpu.VMEM_SHARED((N_PAD, D), jnp.float32),
    ],
)


ROWB = 1000


def _tc1_body(p0, p1, x, wrel, wroot, b, h_ref):
    aggr = p0[...] + p1[...]
    h_ref[...] = (
        jnp.dot(aggr, wrel[...], preferred_element_type=jnp.float32)
        + jnp.dot(x[...], wroot[...], preferred_element_type=jnp.float32)
        + b[...]
    )


def _tc2_body(p0, p1, h, wrel, wroot, b2, wlin, blin, out_ref):
    aggr = p0[...] + p1[...]
    h2 = (
        jnp.dot(aggr, wrel[...], preferred_element_type=jnp.float32)
        + jnp.dot(h[...], wroot[...], preferred_element_type=jnp.float32)
        + b2[...]
    )
    out_ref[...] = jnp.dot(h2, wlin[...], preferred_element_type=jnp.float32) + blin[...]


def _row_spec(d):
    return pl.BlockSpec((ROWB, d), lambda i: (i, 0))


def _full_spec(r, c):
    return pl.BlockSpec((r, c), lambda i: (0, 0))


_tc1 = pl.pallas_call(
    _tc1_body,
    grid=(N // ROWB,),
    in_specs=[
        _row_spec(D), _row_spec(D), _row_spec(D),
        _full_spec(D, D), _full_spec(D, D), _full_spec(1, D),
    ],
    out_specs=_row_spec(D),
    out_shape=jax.ShapeDtypeStruct((N, D), jnp.float32),
)

_tc2 = pl.pallas_call(
    _tc2_body,
    grid=(N // ROWB,),
    in_specs=[
        _row_spec(D), _row_spec(D), _row_spec(D),
        _full_spec(D, D), _full_spec(D, D), _full_spec(1, D),
        _full_spec(D, 64), _full_spec(1, 64),
    ],
    out_specs=_row_spec(64),
    out_shape=jax.ShapeDtypeStruct((N, 64), jnp.float32),
)


def kernel(x, edge_index, W1_rel, b1, W1_root, W2_rel, b2, W2_root, W_lin, b_lin):
    src = edge_index[0].astype(jnp.int32)
    dst = edge_index[1].astype(jnp.int32)

    p = _segsum(x, src, dst)
    h = _tc1(p[0], p[1], x, W1_rel, W1_root, b1.reshape(1, D))
    q = _segsum(h, src, dst)
    out = _tc2(q[0], q[1], h, W2_rel, W2_root, b2.reshape(1, D),
               W_lin, b_lin.reshape(1, 64))
    return out


# speedup vs baseline: 9.7886x; 1.9340x over previous
import jax
import jax.numpy as jnp
from jax import lax
from jax.experimental import pallas as pl
from jax.experimental.pallas import tpu as pltpu
from jax.experimental.pallas import tpu_sc as plsc

N = 10000
E = 320000
D = 128

NC = 2
NS = 16
NW = NC * NS
EPW = E // NW
CHUNK = 40
NCHUNK = EPW // CHUNK
N_PAD = 10240
RPT = N_PAD // NS
NB = 5
NIB = 10
GROUP = 10
PREF_G = 2
PREF_I = 6


def _segsum_body(x_hbm, src_hbm, dst_hbm, out_hbm,
                 src_v, dst_v, rows_v, acc_sh, *sems):
    gsem = sems[:NB]
    ssem = sems[NB:2 * NB]
    isem = sems[2 * NB:2 * NB + NIB]
    jsem = sems[2 * NB + NIB:]
    core = lax.axis_index("c")
    sid = lax.axis_index("s")
    base = (core * NS + sid) * EPW

    def iload(k, bi):
        off = base + k * CHUNK
        pltpu.async_copy(src_hbm.at[pl.ds(off, CHUNK)], src_v.at[bi], isem[bi])
        pltpu.async_copy(dst_hbm.at[pl.ds(off, CHUNK)], dst_v.at[bi], jsem[bi])

    def iload_wait(k, bi):
        off = base + k * CHUNK
        pltpu.make_async_copy(src_hbm.at[pl.ds(off, CHUNK)], src_v.at[bi],
                              isem[bi]).wait()
        pltpu.make_async_copy(dst_hbm.at[pl.ds(off, CHUNK)], dst_v.at[bi],
                              jsem[bi]).wait()

    def gather(ri, bi):
        pltpu.async_copy(x_hbm.at[src_v.at[bi]], rows_v.at[ri], gsem[ri])

    def gather_wait(ri, bi):
        pltpu.make_async_copy(x_hbm.at[src_v.at[bi]], rows_v.at[ri],
                              gsem[ri]).wait()

    def scatter(ri, bi):
        pltpu.async_copy(rows_v.at[ri], acc_sh.at[dst_v.at[bi]], ssem[ri],
                         add=True)

    def scatter_wait(ri, bi):
        pltpu.make_async_copy(rows_v.at[ri], acc_sh.at[dst_v.at[bi]],
                              ssem[ri]).wait()

    for k in range(PREF_I):
        iload(k, k)

    def zrow(i, _):
        for j in range(D // 16):
            rows_v[0, i, pl.ds(j * 16, 16)] = jnp.zeros((16,), jnp.float32)
        return 0
    lax.fori_loop(0, CHUNK, zrow, 0)
    for k in range(RPT // CHUNK):
        pltpu.sync_copy(rows_v.at[0], acc_sh.at[pl.ds(sid * RPT + k * CHUNK, CHUNK)])

    for i in range(PREF_G):
        iload_wait(i, i)
        gather(i, i)

    plsc.subcore_barrier()

    def group(g, _):
        for b in range(GROUP):
            i = g * GROUP + b
            r = b % NB
            gather_wait(r, b)
            scatter(r, b)

            j = i + PREF_G
            rj = (b + PREF_G) % NB
            ij = (b + PREF_G) % NIB
            pj = (b + PREF_G + NB) % NIB

            @pl.when(jnp.logical_and(j >= NB, j < NCHUNK))
            def _():
                scatter_wait(rj, pj)
                iload_wait(j, ij)
                gather(rj, ij)

            @pl.when(j < NB)
            def _():
                iload_wait(j, ij)
                gather(rj, ij)

            k = i + PREF_I
            bk = (b + PREF_I) % NIB

            @pl.when(k < NCHUNK)
            def _():
                iload(k, bk)
        return 0
    lax.fori_loop(0, NCHUNK // GROUP, group, 0)

    for b in range(NB):
        scatter_wait(b, NB + b)

    plsc.subcore_barrier()
    for k in range(RPT // CHUNK):
        r = sid * RPT + k * CHUNK
        pltpu.sync_copy(acc_sh.at[pl.ds(r, CHUNK)], out_hbm.at[core, pl.ds(r, CHUNK)])


_segsum = pl.kernel(
    _segsum_body,
    out_type=jax.ShapeDtypeStruct((NC, N_PAD, D), jnp.float32),
    mesh=plsc.VectorSubcoreMesh(
        core_axis_name="c", subcore_axis_name="s", num_cores=NC, num_subcores=NS
    ),
    scratch_types=[
        pltpu.VMEM((NIB, CHUNK), jnp.int32),
        pltpu.VMEM((NIB, CHUNK), jnp.int32),
        pltpu.VMEM((NB, CHUNK, D), jnp.float32),
        pltpu.VMEM_SHARED((N_PAD, D), jnp.float32),
    ] + [pltpu.SemaphoreType.DMA] * (2 * NB + 2 * NIB),
)


ROWB = 1000


def _tc1_body(p0, p1, x, wrel, wroot, b, h_ref):
    aggr = p0[...] + p1[...]
    h_ref[...] = (
        jnp.dot(aggr, wrel[...], preferred_element_type=jnp.float32)
        + jnp.dot(x[...], wroot[...], preferred_element_type=jnp.float32)
        + b[...]
    )


def _tc2_body(p0, p1, h, wrel, wroot, b2, wlin, blin, out_ref):
    aggr = p0[...] + p1[...]
    h2 = (
        jnp.dot(aggr, wrel[...], preferred_element_type=jnp.float32)
        + jnp.dot(h[...], wroot[...], preferred_element_type=jnp.float32)
        + b2[...]
    )
    out_ref[...] = jnp.dot(h2, wlin[...], preferred_element_type=jnp.float32) + blin[...]


def _row_spec(d):
    return pl.BlockSpec((ROWB, d), lambda i: (i, 0))


def _full_spec(r, c):
    return pl.BlockSpec((r, c), lambda i: (0, 0))


_tc1 = pl.pallas_call(
    _tc1_body,
    grid=(N // ROWB,),
    in_specs=[
        _row_spec(D), _row_spec(D), _row_spec(D),
        _full_spec(D, D), _full_spec(D, D), _full_spec(1, D),
    ],
    out_specs=_row_spec(D),
    out_shape=jax.ShapeDtypeStruct((N, D), jnp.float32),
)

_tc2 = pl.pallas_call(
    _tc2_body,
    grid=(N // ROWB,),
    in_specs=[
        _row_spec(D), _row_spec(D), _row_spec(D),
        _full_spec(D, D), _full_spec(D, D), _full_spec(1, D),
        _full_spec(D, 64), _full_spec(1, 64),
    ],
    out_specs=_row_spec(64),
    out_shape=jax.ShapeDtypeStruct((N, 64), jnp.float32),
)


def kernel(x, edge_index, W1_rel, b1, W1_root, W2_rel, b2, W2_root, W_lin, b_lin):
    src = edge_index[0].astype(jnp.int32)
    dst = edge_index[1].astype(jnp.int32)

    p = _segsum(x, src, dst)
    h = _tc1(p[0], p[1], x, W1_rel, W1_root, b1.reshape(1, D))
    q = _segsum(h, src, dst)
    out = _tc2(q[0], q[1], h, W2_rel, W2_root, b2.reshape(1, D),
               W_lin, b_lin.reshape(1, 64))
    return out


# speedup vs baseline: 12.1188x; 1.2381x over previous
import jax
import jax.numpy as jnp
from jax import lax
from jax.experimental import pallas as pl
from jax.experimental.pallas import tpu as pltpu
from jax.experimental.pallas import tpu_sc as plsc

N = 10000
E = 320000
D = 128

NC = 2
NS = 16
NW = NC * NS
EPW = E // NW
CHUNK = 40
NCHUNK = EPW // CHUNK
N_PAD = 10240
RPT = N_PAD // NS
NB = 5
NIB = 10
GROUP = 10
PREF_G = 3
PREF_I = 6


def _segsum_body(x_hbm, src_hbm, dst_hbm, out_hbm,
                 src_v, dst_v, rows_v, acc_sh, *sems):
    gsem = sems[:NB]
    ssem = sems[NB:2 * NB]
    isem = sems[2 * NB:2 * NB + NIB]
    jsem = sems[2 * NB + NIB:]
    core = lax.axis_index("c")
    sid = lax.axis_index("s")
    base = (core * NS + sid) * EPW

    def iload(k, bi):
        off = base + k * CHUNK
        pltpu.async_copy(src_hbm.at[pl.ds(off, CHUNK)], src_v.at[bi], isem[bi])
        pltpu.async_copy(dst_hbm.at[pl.ds(off, CHUNK)], dst_v.at[bi], jsem[bi])

    def iload_wait(k, bi):
        off = base + k * CHUNK
        pltpu.make_async_copy(src_hbm.at[pl.ds(off, CHUNK)], src_v.at[bi],
                              isem[bi]).wait()
        pltpu.make_async_copy(dst_hbm.at[pl.ds(off, CHUNK)], dst_v.at[bi],
                              jsem[bi]).wait()

    def gather(ri, bi):
        pltpu.async_copy(x_hbm.at[src_v.at[bi]], rows_v.at[ri], gsem[ri])

    def gather_wait(ri, bi):
        pltpu.make_async_copy(x_hbm.at[src_v.at[bi]], rows_v.at[ri],
                              gsem[ri]).wait()

    def scatter(ri, bi):
        pltpu.async_copy(rows_v.at[ri], acc_sh.at[dst_v.at[bi]], ssem[ri],
                         add=True)

    def scatter_wait(ri, bi):
        pltpu.make_async_copy(rows_v.at[ri], acc_sh.at[dst_v.at[bi]],
                              ssem[ri]).wait()

    for k in range(PREF_I):
        iload(k, k)

    def zrow(i, _):
        for j in range(D // 16):
            rows_v[0, i, pl.ds(j * 16, 16)] = jnp.zeros((16,), jnp.float32)
        return 0
    lax.fori_loop(0, CHUNK, zrow, 0)
    for k in range(RPT // CHUNK):
        pltpu.sync_copy(rows_v.at[0], acc_sh.at[pl.ds(sid * RPT + k * CHUNK, CHUNK)])

    for i in range(PREF_G):
        iload_wait(i, i)
        gather(i, i)

    plsc.subcore_barrier()

    def group(g, _):
        for b in range(GROUP):
            i = g * GROUP + b
            r = b % NB
            gather_wait(r, b)
            scatter(r, b)

            j = i + PREF_G
            rj = (b + PREF_G) % NB
            ij = (b + PREF_G) % NIB
            pj = (b + PREF_G + NB) % NIB

            @pl.when(jnp.logical_and(j >= NB, j < NCHUNK))
            def _():
                scatter_wait(rj, pj)
                iload_wait(j, ij)
                gather(rj, ij)

            @pl.when(j < NB)
            def _():
                iload_wait(j, ij)
                gather(rj, ij)

            k = i + PREF_I
            bk = (b + PREF_I) % NIB

            @pl.when(k < NCHUNK)
            def _():
                iload(k, bk)
        return 0
    lax.fori_loop(0, NCHUNK // GROUP, group, 0)

    for b in range(NB):
        scatter_wait(b, NB + b)

    plsc.subcore_barrier()
    for k in range(RPT // CHUNK):
        r = sid * RPT + k * CHUNK
        pltpu.sync_copy(acc_sh.at[pl.ds(r, CHUNK)], out_hbm.at[core, pl.ds(r, CHUNK)])


_segsum = pl.kernel(
    _segsum_body,
    out_type=jax.ShapeDtypeStruct((NC, N_PAD, D), jnp.float32),
    mesh=plsc.VectorSubcoreMesh(
        core_axis_name="c", subcore_axis_name="s", num_cores=NC, num_subcores=NS
    ),
    scratch_types=[
        pltpu.VMEM((NIB, CHUNK), jnp.int32),
        pltpu.VMEM((NIB, CHUNK), jnp.int32),
        pltpu.VMEM((NB, CHUNK, D), jnp.float32),
        pltpu.VMEM_SHARED((N_PAD, D), jnp.float32),
    ] + [pltpu.SemaphoreType.DMA] * (2 * NB + 2 * NIB),
)


ROWB = 1000


def _tc1_body(p0, p1, x, wrel, wroot, b, h_ref):
    aggr = p0[...] + p1[...]
    h_ref[...] = (
        jnp.dot(aggr, wrel[...], preferred_element_type=jnp.float32)
        + jnp.dot(x[...], wroot[...], preferred_element_type=jnp.float32)
        + b[...]
    )


def _tc2_body(p0, p1, h, wrel, wroot, b2, wlin, blin, out_ref):
    aggr = p0[...] + p1[...]
    h2 = (
        jnp.dot(aggr, wrel[...], preferred_element_type=jnp.float32)
        + jnp.dot(h[...], wroot[...], preferred_element_type=jnp.float32)
        + b2[...]
    )
    out_ref[...] = jnp.dot(h2, wlin[...], preferred_element_type=jnp.float32) + blin[...]


def _row_spec(d):
    return pl.BlockSpec((ROWB, d), lambda i: (i, 0))


def _full_spec(r, c):
    return pl.BlockSpec((r, c), lambda i: (0, 0))


_tc1 = pl.pallas_call(
    _tc1_body,
    grid=(N // ROWB,),
    in_specs=[
        _row_spec(D), _row_spec(D), _row_spec(D),
        _full_spec(D, D), _full_spec(D, D), _full_spec(1, D),
    ],
    out_specs=_row_spec(D),
    out_shape=jax.ShapeDtypeStruct((N, D), jnp.float32),
)

_tc2 = pl.pallas_call(
    _tc2_body,
    grid=(N // ROWB,),
    in_specs=[
        _row_spec(D), _row_spec(D), _row_spec(D),
        _full_spec(D, D), _full_spec(D, D), _full_spec(1, D),
        _full_spec(D, 64), _full_spec(1, 64),
    ],
    out_specs=_row_spec(64),
    out_shape=jax.ShapeDtypeStruct((N, 64), jnp.float32),
)


def kernel(x, edge_index, W1_rel, b1, W1_root, W2_rel, b2, W2_root, W_lin, b_lin):
    src = edge_index[0].astype(jnp.int32)
    dst = edge_index[1].astype(jnp.int32)

    p = _segsum(x, src, dst)
    h = _tc1(p[0], p[1], x, W1_rel, W1_root, b1.reshape(1, D))
    q = _segsum(h, src, dst)
    out = _tc2(q[0], q[1], h, W2_rel, W2_root, b2.reshape(1, D),
               W_lin, b_lin.reshape(1, 64))
    return out


# speedup vs baseline: 12.7009x; 1.0480x over previous
import jax
import jax.numpy as jnp
from jax import lax
from jax.experimental import pallas as pl
from jax.experimental.pallas import tpu as pltpu
from jax.experimental.pallas import tpu_sc as plsc

N = 10000
E = 320000
D = 128

NC = 2
NS = 16
NW = NC * NS
EPW = E // NW
CHUNK = 40
NCHUNK = EPW // CHUNK
N_PAD = 10240
RPT = N_PAD // NS
NB = 5
NIB = 10
GROUP = 10
PREF_G = 4
PREF_I = 7


def _segsum_body(x_hbm, src_hbm, dst_hbm, out_hbm,
                 src_v, dst_v, rows_v, acc_sh, *sems):
    gsem = sems[:NB]
    ssem = sems[NB:2 * NB]
    isem = sems[2 * NB:2 * NB + NIB]
    jsem = sems[2 * NB + NIB:]
    core = lax.axis_index("c")
    sid = lax.axis_index("s")
    base = (core * NS + sid) * EPW

    def iload(k, bi):
        off = base + k * CHUNK
        pltpu.async_copy(src_hbm.at[pl.ds(off, CHUNK)], src_v.at[bi], isem[bi])
        pltpu.async_copy(dst_hbm.at[pl.ds(off, CHUNK)], dst_v.at[bi], jsem[bi])

    def iload_wait(k, bi):
        off = base + k * CHUNK
        pltpu.make_async_copy(src_hbm.at[pl.ds(off, CHUNK)], src_v.at[bi],
                              isem[bi]).wait()
        pltpu.make_async_copy(dst_hbm.at[pl.ds(off, CHUNK)], dst_v.at[bi],
                              jsem[bi]).wait()

    def gather(ri, bi):
        pltpu.async_copy(x_hbm.at[src_v.at[bi]], rows_v.at[ri], gsem[ri])

    def gather_wait(ri, bi):
        pltpu.make_async_copy(x_hbm.at[src_v.at[bi]], rows_v.at[ri],
                              gsem[ri]).wait()

    def scatter(ri, bi):
        pltpu.async_copy(rows_v.at[ri], acc_sh.at[dst_v.at[bi]], ssem[ri],
                         add=True)

    def scatter_wait(ri, bi):
        pltpu.make_async_copy(rows_v.at[ri], acc_sh.at[dst_v.at[bi]],
                              ssem[ri]).wait()

    for k in range(PREF_I):
        iload(k, k)

    def zrow(i, _):
        for j in range(D // 16):
            rows_v[0, i, pl.ds(j * 16, 16)] = jnp.zeros((16,), jnp.float32)
        return 0
    lax.fori_loop(0, CHUNK, zrow, 0)
    for k in range(RPT // CHUNK):
        pltpu.sync_copy(rows_v.at[0], acc_sh.at[pl.ds(sid * RPT + k * CHUNK, CHUNK)])

    for i in range(PREF_G):
        iload_wait(i, i)
        gather(i, i)

    plsc.subcore_barrier()

    def group(g, _):
        for b in range(GROUP):
            i = g * GROUP + b
            r = b % NB
            gather_wait(r, b)
            scatter(r, b)

            j = i + PREF_G
            rj = (b + PREF_G) % NB
            ij = (b + PREF_G) % NIB
            pj = (b + PREF_G + NB) % NIB

            @pl.when(jnp.logical_and(j >= NB, j < NCHUNK))
            def _():
                scatter_wait(rj, pj)
                iload_wait(j, ij)
                gather(rj, ij)

            @pl.when(j < NB)
            def _():
                iload_wait(j, ij)
                gather(rj, ij)

            k = i + PREF_I
            bk = (b + PREF_I) % NIB

            @pl.when(k < NCHUNK)
            def _():
                iload(k, bk)
        return 0
    lax.fori_loop(0, NCHUNK // GROUP, group, 0)

    for b in range(NB):
        scatter_wait(b, NB + b)

    plsc.subcore_barrier()
    for k in range(RPT // CHUNK):
        r = sid * RPT + k * CHUNK
        pltpu.sync_copy(acc_sh.at[pl.ds(r, CHUNK)], out_hbm.at[core, pl.ds(r, CHUNK)])


_segsum = pl.kernel(
    _segsum_body,
    out_type=jax.ShapeDtypeStruct((NC, N_PAD, D), jnp.float32),
    mesh=plsc.VectorSubcoreMesh(
        core_axis_name="c", subcore_axis_name="s", num_cores=NC, num_subcores=NS
    ),
    scratch_types=[
        pltpu.VMEM((NIB, CHUNK), jnp.int32),
        pltpu.VMEM((NIB, CHUNK), jnp.int32),
        pltpu.VMEM((NB, CHUNK, D), jnp.float32),
        pltpu.VMEM_SHARED((N_PAD, D), jnp.float32),
    ] + [pltpu.SemaphoreType.DMA] * (2 * NB + 2 * NIB),
)


ROWB = 1000


def _tc1_body(p0, p1, x, wrel, wroot, b, h_ref):
    aggr = p0[...] + p1[...]
    h_ref[...] = (
        jnp.dot(aggr, wrel[...], preferred_element_type=jnp.float32)
        + jnp.dot(x[...], wroot[...], preferred_element_type=jnp.float32)
        + b[...]
    )


def _tc2_body(p0, p1, h, wrel, wroot, b2, wlin, blin, out_ref):
    aggr = p0[...] + p1[...]
    h2 = (
        jnp.dot(aggr, wrel[...], preferred_element_type=jnp.float32)
        + jnp.dot(h[...], wroot[...], preferred_element_type=jnp.float32)
        + b2[...]
    )
    out_ref[...] = jnp.dot(h2, wlin[...], preferred_element_type=jnp.float32) + blin[...]


def _row_spec(d):
    return pl.BlockSpec((ROWB, d), lambda i: (i, 0))


def _full_spec(r, c):
    return pl.BlockSpec((r, c), lambda i: (0, 0))


_tc1 = pl.pallas_call(
    _tc1_body,
    grid=(N // ROWB,),
    in_specs=[
        _row_spec(D), _row_spec(D), _row_spec(D),
        _full_spec(D, D), _full_spec(D, D), _full_spec(1, D),
    ],
    out_specs=_row_spec(D),
    out_shape=jax.ShapeDtypeStruct((N, D), jnp.float32),
)

_tc2 = pl.pallas_call(
    _tc2_body,
    grid=(N // ROWB,),
    in_specs=[
        _row_spec(D), _row_spec(D), _row_spec(D),
        _full_spec(D, D), _full_spec(D, D), _full_spec(1, D),
        _full_spec(D, 64), _full_spec(1, 64),
    ],
    out_specs=_row_spec(64),
    out_shape=jax.ShapeDtypeStruct((N, 64), jnp.float32),
)


def kernel(x, edge_index, W1_rel, b1, W1_root, W2_rel, b2, W2_root, W_lin, b_lin):
    src = edge_index[0].astype(jnp.int32)
    dst = edge_index[1].astype(jnp.int32)

    p = _segsum(x, src, dst)
    h = _tc1(p[0], p[1], x, W1_rel, W1_root, b1.reshape(1, D))
    q = _segsum(h, src, dst)
    out = _tc2(q[0], q[1], h, W2_rel, W2_root, b2.reshape(1, D),
               W_lin, b_lin.reshape(1, 64))
    return out


# speedup vs baseline: 13.4579x; 1.0596x over previous
import jax
import jax.numpy as jnp
from jax import lax
from jax.experimental import pallas as pl
from jax.experimental.pallas import tpu as pltpu
from jax.experimental.pallas import tpu_sc as plsc

N = 10000
E = 320000
D = 128

NC = 2
NS = 16
NW = NC * NS
EPW = E // NW
CHUNK = 40
NCHUNK = EPW // CHUNK
N_PAD = 10240
RPT = N_PAD // NS
NB = 5
NIB = 10
GROUP = 10
PREF_G = 4
PREF_I = 7


def _segsum_body(x_hbm, src_hbm, dst_hbm, out_hbm,
                 src_v, dst_v, rows_v, acc_sh, *sems):
    gsem = sems[:NB]
    ssem = sems[NB:2 * NB]
    isem = sems[2 * NB:2 * NB + NIB]
    jsem = sems[2 * NB + NIB:]
    core = lax.axis_index("c")
    sid = lax.axis_index("s")
    base = (core * NS + sid) * EPW

    def iload(k, bi):
        off = base + k * CHUNK
        pltpu.async_copy(src_hbm.at[pl.ds(off, CHUNK)], src_v.at[bi], isem[bi])
        pltpu.async_copy(dst_hbm.at[pl.ds(off, CHUNK)], dst_v.at[bi], jsem[bi])

    def iload_wait(k, bi):
        off = base + k * CHUNK
        pltpu.make_async_copy(src_hbm.at[pl.ds(off, CHUNK)], src_v.at[bi],
                              isem[bi]).wait()
        pltpu.make_async_copy(dst_hbm.at[pl.ds(off, CHUNK)], dst_v.at[bi],
                              jsem[bi]).wait()

    def gather(ri, bi):
        pltpu.async_copy(x_hbm.at[src_v.at[bi]], rows_v.at[ri], gsem[ri])

    def gather_wait(ri, bi):
        pltpu.make_async_copy(x_hbm.at[src_v.at[bi]], rows_v.at[ri],
                              gsem[ri]).wait()

    def scatter(ri, bi):
        pltpu.async_copy(rows_v.at[ri], acc_sh.at[dst_v.at[bi]], ssem[ri],
                         add=True)

    def scatter_wait(ri, bi):
        pltpu.make_async_copy(rows_v.at[ri], acc_sh.at[dst_v.at[bi]],
                              ssem[ri]).wait()

    for k in range(PREF_I):
        iload(k, k)

    def zrow(i, _):
        for j in range(D // 16):
            rows_v[0, i, pl.ds(j * 16, 16)] = jnp.zeros((16,), jnp.float32)
        return 0
    lax.fori_loop(0, CHUNK, zrow, 0)
    for k in range(RPT // CHUNK):
        pltpu.sync_copy(rows_v.at[0], acc_sh.at[pl.ds(sid * RPT + k * CHUNK, CHUNK)])

    for i in range(PREF_G):
        iload_wait(i, i)
        gather(i, i)

    plsc.subcore_barrier()

    def group(g, _):
        for b in range(GROUP):
            i = g * GROUP + b
            r = b % NB
            gather_wait(r, b)
            scatter(r, b)

            j = i + PREF_G
            rj = (b + PREF_G) % NB
            ij = (b + PREF_G) % NIB
            pj = (b + PREF_G + NB) % NIB

            @pl.when(jnp.logical_and(j >= NB, j < NCHUNK))
            def _():
                scatter_wait(rj, pj)
                iload_wait(j, ij)
                gather(rj, ij)

            @pl.when(j < NB)
            def _():
                iload_wait(j, ij)
                gather(rj, ij)

            k = i + PREF_I
            bk = (b + PREF_I) % NIB

            @pl.when(k < NCHUNK)
            def _():
                iload(k, bk)
        return 0
    lax.fori_loop(0, NCHUNK // GROUP, group, 0)

    for b in range(NB):
        scatter_wait(b, NB + b)

    plsc.subcore_barrier()
    for k in range(RPT // CHUNK):
        r = sid * RPT + k * CHUNK
        pltpu.sync_copy(acc_sh.at[pl.ds(r, CHUNK)], out_hbm.at[core, pl.ds(r, CHUNK)])


_segsum = pl.kernel(
    _segsum_body,
    out_type=jax.ShapeDtypeStruct((NC, N_PAD, D), jnp.float32),
    mesh=plsc.VectorSubcoreMesh(
        core_axis_name="c", subcore_axis_name="s", num_cores=NC, num_subcores=NS
    ),
    scratch_types=[
        pltpu.VMEM((NIB, CHUNK), jnp.int32),
        pltpu.VMEM((NIB, CHUNK), jnp.int32),
        pltpu.VMEM((NB, CHUNK, D), jnp.float32),
        pltpu.VMEM_SHARED((N_PAD, D), jnp.float32),
    ] + [pltpu.SemaphoreType.DMA] * (2 * NB + 2 * NIB),
)


ROWB = 1000


def _tc1_body(p, x, wrel, wroot, b1, wroot2, wlin, b2, blin, h_ref, hw_ref):
    aggr = p[0] + p[1]
    h = (
        jnp.dot(aggr, wrel[...], preferred_element_type=jnp.float32)
        + jnp.dot(x[...], wroot[...], preferred_element_type=jnp.float32)
        + b1[...]
    )
    h_ref[...] = h
    b2w = jnp.dot(wroot2[...], wlin[...], preferred_element_type=jnp.float32)
    hw_ref[...] = (
        jnp.dot(h, b2w, preferred_element_type=jnp.float32)
        + jnp.dot(b2[...], wlin[...], preferred_element_type=jnp.float32)
        + blin[...]
    )


def _tc2_body(q, hw, wrel2, wlin, out_ref):
    aggr = q[0] + q[1]
    a2 = jnp.dot(wrel2[...], wlin[...], preferred_element_type=jnp.float32)
    out_ref[...] = jnp.dot(aggr, a2, preferred_element_type=jnp.float32) + hw[...]


def _row_spec(d):
    return pl.BlockSpec((ROWB, d), lambda i: (i, 0))


def _pair_spec(d):
    return pl.BlockSpec((2, ROWB, d), lambda i: (0, i, 0))


def _full_spec(r, c):
    return pl.BlockSpec((r, c), lambda i: (0, 0))


_tc1 = pl.pallas_call(
    _tc1_body,
    grid=(N // ROWB,),
    in_specs=[
        _pair_spec(D), _row_spec(D),
        _full_spec(D, D), _full_spec(D, D), _full_spec(1, D),
        _full_spec(D, D), _full_spec(D, 64), _full_spec(1, D), _full_spec(1, 64),
    ],
    out_specs=[_row_spec(D), _row_spec(64)],
    out_shape=[
        jax.ShapeDtypeStruct((N, D), jnp.float32),
        jax.ShapeDtypeStruct((N, 64), jnp.float32),
    ],
)

_tc2 = pl.pallas_call(
    _tc2_body,
    grid=(N // ROWB,),
    in_specs=[
        _pair_spec(D), _row_spec(64),
        _full_spec(D, D), _full_spec(D, 64),
    ],
    out_specs=_row_spec(64),
    out_shape=jax.ShapeDtypeStruct((N, 64), jnp.float32),
)


def kernel(x, edge_index, W1_rel, b1, W1_root, W2_rel, b2, W2_root, W_lin, b_lin):
    src = edge_index[0].astype(jnp.int32)
    dst = edge_index[1].astype(jnp.int32)

    p = _segsum(x, src, dst)
    h, hw = _tc1(p, x, W1_rel, W1_root, b1.reshape(1, D),
                 W2_root, W_lin, b2.reshape(1, D), b_lin.reshape(1, 64))
    q = _segsum(h, src, dst)
    out = _tc2(q, hw, W2_rel, W_lin)
    return out


# speedup vs baseline: 13.9959x; 1.0400x over previous
import jax
import jax.numpy as jnp
from jax import lax
from jax.experimental import pallas as pl
from jax.experimental.pallas import tpu as pltpu
from jax.experimental.pallas import tpu_sc as plsc

N = 10000
E = 320000
D = 128

NC = 2
NS = 16
NW = NC * NS
EPW = E // NW
CHUNK = 40
NCHUNK = EPW // CHUNK
N_PAD = 10240
RPT = N_PAD // NS
NB = 5
NIB = 10
GROUP = 10
PREF_G = 4
PREF_I = 7


def _segsum_body(x_hbm, ei_hbm, out_hbm,
                 src_v, dst_v, rows_v, acc_sh, *sems):
    gsem = sems[:NB]
    ssem = sems[NB:2 * NB]
    isem = sems[2 * NB:2 * NB + NIB]
    jsem = sems[2 * NB + NIB:]
    core = lax.axis_index("c")
    sid = lax.axis_index("s")
    base = (core * NS + sid) * EPW

    def iload(k, bi):
        off = base + k * CHUNK
        pltpu.async_copy(ei_hbm.at[pl.ds(off, CHUNK)], src_v.at[bi], isem[bi])
        pltpu.async_copy(ei_hbm.at[pl.ds(E + off, CHUNK)], dst_v.at[bi], jsem[bi])

    def iload_wait(k, bi):
        off = base + k * CHUNK
        pltpu.make_async_copy(ei_hbm.at[pl.ds(off, CHUNK)], src_v.at[bi],
                              isem[bi]).wait()
        pltpu.make_async_copy(ei_hbm.at[pl.ds(E + off, CHUNK)], dst_v.at[bi],
                              jsem[bi]).wait()

    def gather(ri, bi):
        pltpu.async_copy(x_hbm.at[src_v.at[bi]], rows_v.at[ri], gsem[ri])

    def gather_wait(ri, bi):
        pltpu.make_async_copy(x_hbm.at[src_v.at[bi]], rows_v.at[ri],
                              gsem[ri]).wait()

    def scatter(ri, bi):
        pltpu.async_copy(rows_v.at[ri], acc_sh.at[dst_v.at[bi]], ssem[ri],
                         add=True)

    def scatter_wait(ri, bi):
        pltpu.make_async_copy(rows_v.at[ri], acc_sh.at[dst_v.at[bi]],
                              ssem[ri]).wait()

    for k in range(PREF_I):
        iload(k, k)

    def zrow(i, _):
        for j in range(D // 16):
            rows_v[0, i, pl.ds(j * 16, 16)] = jnp.zeros((16,), jnp.float32)
        return 0
    lax.fori_loop(0, CHUNK, zrow, 0)
    for k in range(RPT // CHUNK):
        pltpu.sync_copy(rows_v.at[0], acc_sh.at[pl.ds(sid * RPT + k * CHUNK, CHUNK)])

    for i in range(PREF_G):
        iload_wait(i, i)
        gather(i, i)

    plsc.subcore_barrier()

    def group(g, _):
        for b in range(GROUP):
            i = g * GROUP + b
            r = b % NB
            gather_wait(r, b)
            scatter(r, b)

            j = i + PREF_G
            rj = (b + PREF_G) % NB
            ij = (b + PREF_G) % NIB
            pj = (b + PREF_G + NB) % NIB

            @pl.when(jnp.logical_and(j >= NB, j < NCHUNK))
            def _():
                scatter_wait(rj, pj)
                iload_wait(j, ij)
                gather(rj, ij)

            @pl.when(j < NB)
            def _():
                iload_wait(j, ij)
                gather(rj, ij)

            k = i + PREF_I
            bk = (b + PREF_I) % NIB

            @pl.when(k < NCHUNK)
            def _():
                iload(k, bk)
        return 0
    lax.fori_loop(0, NCHUNK // GROUP, group, 0)

    for b in range(NB):
        scatter_wait(b, NB + b)

    plsc.subcore_barrier()
    for k in range(RPT // CHUNK):
        r = sid * RPT + k * CHUNK
        pltpu.sync_copy(acc_sh.at[pl.ds(r, CHUNK)], out_hbm.at[core, pl.ds(r, CHUNK)])


_segsum = pl.kernel(
    _segsum_body,
    out_type=jax.ShapeDtypeStruct((NC, N_PAD, D), jnp.float32),
    mesh=plsc.VectorSubcoreMesh(
        core_axis_name="c", subcore_axis_name="s", num_cores=NC, num_subcores=NS
    ),
    scratch_types=[
        pltpu.VMEM((NIB, CHUNK), jnp.int32),
        pltpu.VMEM((NIB, CHUNK), jnp.int32),
        pltpu.VMEM((NB, CHUNK, D), jnp.float32),
        pltpu.VMEM_SHARED((N_PAD, D), jnp.float32),
    ] + [pltpu.SemaphoreType.DMA] * (2 * NB + 2 * NIB),
)


ROWB = 1000


def _tc1_body(p, x, wrel, wroot, b1, wroot2, wlin, b2, blin, h_ref, hw_ref):
    aggr = p[0] + p[1]
    h = (
        jnp.dot(aggr, wrel[...], preferred_element_type=jnp.float32)
        + jnp.dot(x[...], wroot[...], preferred_element_type=jnp.float32)
        + b1[...]
    )
    h_ref[...] = h
    b2w = jnp.dot(wroot2[...], wlin[...], preferred_element_type=jnp.float32)
    hw_ref[...] = (
        jnp.dot(h, b2w, preferred_element_type=jnp.float32)
        + jnp.dot(b2[...], wlin[...], preferred_element_type=jnp.float32)
        + blin[...]
    )


def _tc2_body(q, hw, wrel2, wlin, out_ref):
    aggr = q[0] + q[1]
    a2 = jnp.dot(wrel2[...], wlin[...], preferred_element_type=jnp.float32)
    out_ref[...] = jnp.dot(aggr, a2, preferred_element_type=jnp.float32) + hw[...]


def _row_spec(d):
    return pl.BlockSpec((ROWB, d), lambda i: (i, 0))


def _pair_spec(d):
    return pl.BlockSpec((2, ROWB, d), lambda i: (0, i, 0))


def _full_spec(r, c):
    return pl.BlockSpec((r, c), lambda i: (0, 0))


_tc1 = pl.pallas_call(
    _tc1_body,
    grid=(N // ROWB,),
    in_specs=[
        _pair_spec(D), _row_spec(D),
        _full_spec(D, D), _full_spec(D, D), _full_spec(1, D),
        _full_spec(D, D), _full_spec(D, 64), _full_spec(1, D), _full_spec(1, 64),
    ],
    out_specs=[_row_spec(D), _row_spec(64)],
    out_shape=[
        jax.ShapeDtypeStruct((N, D), jnp.float32),
        jax.ShapeDtypeStruct((N, 64), jnp.float32),
    ],
)

_tc2 = pl.pallas_call(
    _tc2_body,
    grid=(N // ROWB,),
    in_specs=[
        _pair_spec(D), _row_spec(64),
        _full_spec(D, D), _full_spec(D, 64),
    ],
    out_specs=_row_spec(64),
    out_shape=jax.ShapeDtypeStruct((N, 64), jnp.float32),
)


def kernel(x, edge_index, W1_rel, b1, W1_root, W2_rel, b2, W2_root, W_lin, b_lin):
    ei = edge_index.astype(jnp.int32).reshape(2 * E)

    p = _segsum(x, ei)
    h, hw = _tc1(p, x, W1_rel, W1_root, b1.reshape(1, D),
                 W2_root, W_lin, b2.reshape(1, D), b_lin.reshape(1, 64))
    q = _segsum(h, ei)
    out = _tc2(q, hw, W2_rel, W_lin)
    return out
